# baseline (speedup 1.0000x reference)
import functools

import jax
import jax.numpy as jnp
from jax import lax
from jax.experimental import pallas as pl
from jax.experimental.pallas import tpu as pltpu
from jax.experimental.pallas import tpu_sc as plsc

BATCH = 16384
FEAT = 32
NC = 2
NS = 16
LANES = 16
NGROUPS = BATCH // LANES
CHUNK = 32768


def _body(idx_hbm, t0_hbm, t1_hbm, t2_hbm, t3_hbm, out_hbm,
          idx_v, acc, p0, p1, bx, by, sem_a, sem_x, sem_y):
    c = lax.axis_index("s") * NC + lax.axis_index("c")

    bufs = (bx, by)
    sems = (sem_x, sem_y)
    passes = [(t2_hbm, k, bufs[k % 2], sems[k % 2]) for k in range(2)]
    passes += [(t3_hbm, k, bufs[k % 2], sems[k % 2]) for k in range(8)]

    def start(i):
        tbl, k, buf, sem = passes[i]
        return pltpu.async_copy(tbl.at[c, pl.ds(k * CHUNK, CHUNK)], buf, sem)

    ci = pltpu.async_copy(idx_hbm, idx_v, sem_a)
    c0 = pltpu.async_copy(t0_hbm.at[c], p0, sem_a)
    c1 = pltpu.async_copy(t1_hbm.at[c], p1, sem_a)
    d0 = start(0)
    d1 = start(1)
    dmas = [d0, d1]
    ci.wait()
    c0.wait()
    c1.wait()

    def phase_a(j, _):
        s = pl.ds(j * LANES, LANES)
        v = idx_v[s]
        a = plsc.load_gather(p0, [lax.bitwise_and(v, 4095)])
        b = plsc.load_gather(p1, [lax.bitwise_and(v, 16383)])
        acc[s] = a + b
        return 0

    lax.fori_loop(0, NGROUPS, phase_a, 0, unroll=4)

    for i in range(10):
        _, k, buf, _ = passes[i]
        dmas[i % 2].wait()
        is_lod2 = i < 2

        def chunk_pass(j, _, buf=buf, k=k, is_lod2=is_lod2):
            s = pl.ds(j * LANES, LANES)
            v = idx_v[s]
            key = lax.shift_right_logical(v, 15)
            if is_lod2:
                key = lax.bitwise_and(key, 1)
            val = plsc.load_gather(buf, [lax.bitwise_and(v, CHUNK - 1)])
            hit = lax.eq(key, k)
            plsc.addupdate(acc.at[s], jnp.where(hit, val, 0.0))
            return 0

        lax.fori_loop(0, NGROUPS, chunk_pass, 0, unroll=4)

        if i + 2 < 10:
            dmas[i % 2] = start(i + 2)

    pltpu.sync_copy(acc, out_hbm.at[c])


@jax.jit
def _octree_lookup(indices, cb0, cb1, cb2, cb3):
    ts = [cb.T for cb in (cb0, cb1, cb2, cb3)]
    mesh = plsc.VectorSubcoreMesh(core_axis_name="c", subcore_axis_name="s")
    f = functools.partial(
        pl.kernel,
        mesh=mesh,
        compiler_params=pltpu.CompilerParams(needs_layout_passes=False),
        out_type=jax.ShapeDtypeStruct((FEAT, BATCH), jnp.float32),
        scratch_types=[
            pltpu.VMEM((BATCH,), jnp.int32),
            pltpu.VMEM((BATCH,), jnp.float32),
            pltpu.VMEM((4096,), jnp.float32),
            pltpu.VMEM((16384,), jnp.float32),
            pltpu.VMEM((CHUNK,), jnp.float32),
            pltpu.VMEM((CHUNK,), jnp.float32),
            pltpu.SemaphoreType.DMA,
            pltpu.SemaphoreType.DMA,
            pltpu.SemaphoreType.DMA,
        ],
    )(_body)
    out_t = f(indices, *ts)
    return out_t.T


def kernel(indices, cb0, cb1, cb2, cb3):
    return _octree_lookup(indices.astype(jnp.int32), cb0, cb1, cb2, cb3)


# speedup vs baseline: 1.9313x; 1.9313x over previous
import functools

import jax
import jax.numpy as jnp
from jax import lax
from jax.experimental import pallas as pl
from jax.experimental.pallas import tpu as pltpu
from jax.experimental.pallas import tpu_sc as plsc

BATCH = 16384
FEAT = 32
NC = 2
NS = 16
LANES = 16
NGROUPS = BATCH // LANES
CHUNK = 65536


def _body(idx_hbm, t0_hbm, t1_hbm, t2_hbm, t3_hbm, out_hbm,
          idx_v, acc, p0, p1, pbuf, sem):
    c = lax.axis_index("s") * NC + lax.axis_index("c")

    ci = pltpu.async_copy(idx_hbm, idx_v, sem)
    c0 = pltpu.async_copy(t0_hbm.at[c], p0, sem)
    c1 = pltpu.async_copy(t1_hbm.at[c], p1, sem)
    c2 = pltpu.async_copy(t2_hbm.at[c], pbuf, sem)
    ci.wait()
    c0.wait()
    c1.wait()
    c2.wait()

    def phase_a(j, _):
        s = pl.ds(j * LANES, LANES)
        v = idx_v[s]
        a = plsc.load_gather(p0, [lax.bitwise_and(v, 4095)])
        b = plsc.load_gather(p1, [lax.bitwise_and(v, 16383)])
        d = plsc.load_gather(pbuf, [lax.bitwise_and(v, 65535)])
        acc[s] = (a + b) + d
        return 0

    lax.fori_loop(0, NGROUPS, phase_a, 0, unroll=8)

    for k in range(4):
        ck = pltpu.async_copy(t3_hbm.at[c, pl.ds(k * CHUNK, CHUNK)], pbuf, sem)
        ck.wait()

        def phase_b(j, _, k=k):
            s = pl.ds(j * LANES, LANES)
            v = idx_v[s]
            val = plsc.load_gather(pbuf, [lax.bitwise_and(v, 65535)])
            hit = lax.eq(lax.shift_right_logical(v, 16), k)
            plsc.addupdate(acc.at[s], jnp.where(hit, val, 0.0))
            return 0

        lax.fori_loop(0, NGROUPS, phase_b, 0, unroll=8)

    pltpu.sync_copy(acc, out_hbm.at[c])


@jax.jit
def _octree_lookup(indices, cb0, cb1, cb2, cb3):
    ts = [cb.T for cb in (cb0, cb1, cb2, cb3)]
    mesh = plsc.VectorSubcoreMesh(core_axis_name="c", subcore_axis_name="s")
    f = functools.partial(
        pl.kernel,
        mesh=mesh,
        compiler_params=pltpu.CompilerParams(needs_layout_passes=False),
        out_type=jax.ShapeDtypeStruct((FEAT, BATCH), jnp.float32),
        scratch_types=[
            pltpu.VMEM((BATCH,), jnp.int32),
            pltpu.VMEM((BATCH,), jnp.float32),
            pltpu.VMEM((4096,), jnp.float32),
            pltpu.VMEM((16384,), jnp.float32),
            pltpu.VMEM((CHUNK,), jnp.float32),
            pltpu.SemaphoreType.DMA,
        ],
    )(_body)
    out_t = f(indices, *ts)
    return out_t.T


def kernel(indices, cb0, cb1, cb2, cb3):
    return _octree_lookup(indices.astype(jnp.int32), cb0, cb1, cb2, cb3)


# speedup vs baseline: 1.9626x; 1.0162x over previous
import functools

import jax
import jax.numpy as jnp
from jax import lax
from jax.experimental import pallas as pl
from jax.experimental.pallas import tpu as pltpu
from jax.experimental.pallas import tpu_sc as plsc

BATCH = 16384
FEAT = 32
NC = 2
NS = 16
LANES = 16
NGROUPS = BATCH // LANES
CHUNK = 65536


def _body(idx_hbm, t0_hbm, t1_hbm, t2_hbm, t3_hbm, out_hbm,
          idx_v, acc, p0, p1, pbuf, sem):
    c = lax.axis_index("s") * NC + lax.axis_index("c")

    ci = pltpu.async_copy(idx_hbm, idx_v, sem)
    c0 = pltpu.async_copy(t0_hbm.at[c], p0, sem)
    c1 = pltpu.async_copy(t1_hbm.at[c], p1, sem)
    c2 = pltpu.async_copy(t2_hbm.at[c], pbuf, sem)
    ci.wait()
    c0.wait()
    c1.wait()
    c2.wait()

    def phase_a(j, _):
        s = pl.ds(j * LANES, LANES)
        v = idx_v[s]
        a = plsc.load_gather(p0, [lax.bitwise_and(v, 4095)])
        b = plsc.load_gather(p1, [lax.bitwise_and(v, 16383)])
        d = plsc.load_gather(pbuf, [lax.bitwise_and(v, 65535)])
        acc[s] = (a + b) + d
        return 0

    lax.fori_loop(0, NGROUPS, phase_a, 0, unroll=8)

    for k in range(4):
        ck = pltpu.async_copy(t3_hbm.at[c, pl.ds(k * CHUNK, CHUNK)], pbuf, sem)
        ck.wait()

        def phase_b(j, _, k=k):
            s = pl.ds(j * LANES, LANES)
            v = idx_v[s]
            hit = lax.eq(lax.shift_right_logical(v, 16), k)
            val = plsc.load_gather(
                pbuf, [lax.bitwise_and(v, 65535)], mask=hit)
            plsc.addupdate(acc.at[s], jnp.where(hit, val, 0.0))
            return 0

        lax.fori_loop(0, NGROUPS, phase_b, 0, unroll=8)

    pltpu.sync_copy(acc, out_hbm.at[c])


@jax.jit
def _octree_lookup(indices, cb0, cb1, cb2, cb3):
    ts = [cb.T for cb in (cb0, cb1, cb2, cb3)]
    mesh = plsc.VectorSubcoreMesh(core_axis_name="c", subcore_axis_name="s")
    f = functools.partial(
        pl.kernel,
        mesh=mesh,
        compiler_params=pltpu.CompilerParams(needs_layout_passes=False),
        out_type=jax.ShapeDtypeStruct((FEAT, BATCH), jnp.float32),
        scratch_types=[
            pltpu.VMEM((BATCH,), jnp.int32),
            pltpu.VMEM((BATCH,), jnp.float32),
            pltpu.VMEM((4096,), jnp.float32),
            pltpu.VMEM((16384,), jnp.float32),
            pltpu.VMEM((CHUNK,), jnp.float32),
            pltpu.SemaphoreType.DMA,
        ],
    )(_body)
    out_t = f(indices, *ts)
    return out_t.T


def kernel(indices, cb0, cb1, cb2, cb3):
    return _octree_lookup(indices.astype(jnp.int32), cb0, cb1, cb2, cb3)


# speedup vs baseline: 2.8598x; 1.4571x over previous
import functools

import jax
import jax.numpy as jnp
from jax import lax
from jax.experimental import pallas as pl
from jax.experimental.pallas import tpu as pltpu
from jax.experimental.pallas import tpu_sc as plsc

BATCH = 16384
FEAT = 32
NC = 2
NS = 16
LANES = 16
NGROUPS = BATCH // LANES
CHUNK = 65536


def _body(idx_hbm, t0_hbm, t1_hbm, t2_hbm, t3_hbm, out_hbm,
          idx_v, acc, p0, p1, pbuf, sem):
    c = lax.axis_index("s") * NC + lax.axis_index("c")

    ci = pltpu.async_copy(idx_hbm, idx_v, sem)
    c0 = pltpu.async_copy(t0_hbm.at[c], p0, sem)
    c1 = pltpu.async_copy(t1_hbm.at[c], p1, sem)
    c2 = pltpu.async_copy(t2_hbm.at[c], pbuf, sem)
    ci.wait()
    c0.wait()
    c1.wait()
    c2.wait()

    @plsc.parallel_loop(0, BATCH, LANES, unroll=8)
    def _(i):
        s = pl.ds(i, LANES)
        v = idx_v[s]
        a = plsc.load_gather(p0, [lax.bitwise_and(v, 4095)])
        b = plsc.load_gather(p1, [lax.bitwise_and(v, 16383)])
        d = plsc.load_gather(pbuf, [lax.bitwise_and(v, 65535)])
        acc[s] = (a + b) + d

    for k in range(4):
        ck = pltpu.async_copy(t3_hbm.at[c, pl.ds(k * CHUNK, CHUNK)], pbuf, sem)
        ck.wait()

        @plsc.parallel_loop(0, BATCH, LANES, unroll=8)
        def _(i, k=k):
            s = pl.ds(i, LANES)
            v = idx_v[s]
            hit = lax.eq(lax.shift_right_logical(v, 16), k)
            val = plsc.load_gather(
                pbuf, [lax.bitwise_and(v, 65535)], mask=hit)
            plsc.addupdate(acc.at[s], jnp.where(hit, val, 0.0))

    pltpu.sync_copy(acc, out_hbm.at[c])


@jax.jit
def _octree_lookup(indices, cb0, cb1, cb2, cb3):
    ts = [cb.T for cb in (cb0, cb1, cb2, cb3)]
    mesh = plsc.VectorSubcoreMesh(core_axis_name="c", subcore_axis_name="s")
    f = functools.partial(
        pl.kernel,
        mesh=mesh,
        compiler_params=pltpu.CompilerParams(needs_layout_passes=False),
        out_type=jax.ShapeDtypeStruct((FEAT, BATCH), jnp.float32),
        scratch_types=[
            pltpu.VMEM((BATCH,), jnp.int32),
            pltpu.VMEM((BATCH,), jnp.float32),
            pltpu.VMEM((4096,), jnp.float32),
            pltpu.VMEM((16384,), jnp.float32),
            pltpu.VMEM((CHUNK,), jnp.float32),
            pltpu.SemaphoreType.DMA,
        ],
    )(_body)
    out_t = f(indices, *ts)
    return out_t.T


def kernel(indices, cb0, cb1, cb2, cb3):
    return _octree_lookup(indices.astype(jnp.int32), cb0, cb1, cb2, cb3)


# speedup vs baseline: 2.9570x; 1.0340x over previous
import functools

import jax
import jax.numpy as jnp
from jax import lax
from jax.experimental import pallas as pl
from jax.experimental.pallas import tpu as pltpu
from jax.experimental.pallas import tpu_sc as plsc

BATCH = 16384
FEAT = 32
NC = 2
NS = 16
LANES = 16
NGROUPS = BATCH // LANES
CHUNK = 32768


def _body(idx_hbm, t0_hbm, t1_hbm, t2_hbm, t3_hbm, out_hbm,
          idx_v, acc, p0, p1, bx, by, sem_a, sem_x, sem_y):
    c = lax.axis_index("s") * NC + lax.axis_index("c")

    bufs = (bx, by)
    sems = (sem_x, sem_y)
    passes = [(t2_hbm, k, True) for k in range(2)]
    passes += [(t3_hbm, k, False) for k in range(8)]

    def start(i):
        tbl, k, _ = passes[i]
        return pltpu.async_copy(
            tbl.at[c, pl.ds(k * CHUNK, CHUNK)], bufs[i % 2], sems[i % 2])

    ci = pltpu.async_copy(idx_hbm, idx_v, sem_a)
    c0 = pltpu.async_copy(t0_hbm.at[c], p0, sem_a)
    c1 = pltpu.async_copy(t1_hbm.at[c], p1, sem_a)
    dmas = [start(0), start(1)]
    ci.wait()
    c0.wait()
    c1.wait()

    @plsc.parallel_loop(0, BATCH, LANES, unroll=8)
    def _(i):
        s = pl.ds(i, LANES)
        v = idx_v[s]
        a = plsc.load_gather(p0, [lax.bitwise_and(v, 4095)])
        b = plsc.load_gather(p1, [lax.bitwise_and(v, 16383)])
        acc[s] = a + b

    for i in range(10):
        _, k, is_lod2 = passes[i]
        buf = bufs[i % 2]
        dmas[i % 2].wait()

        @plsc.parallel_loop(0, BATCH, LANES, unroll=8)
        def _(q, buf=buf, k=k, is_lod2=is_lod2):
            s = pl.ds(q, LANES)
            v = idx_v[s]
            key = lax.shift_right_logical(v, 15)
            if is_lod2:
                key = lax.bitwise_and(key, 1)
            hit = lax.eq(key, k)
            val = plsc.load_gather(
                buf, [lax.bitwise_and(v, CHUNK - 1)], mask=hit)
            plsc.addupdate(acc.at[s], jnp.where(hit, val, 0.0))

        if i + 2 < 10:
            dmas[i % 2] = start(i + 2)

    pltpu.sync_copy(acc, out_hbm.at[c])


@jax.jit
def _octree_lookup(indices, cb0, cb1, cb2, cb3):
    ts = [cb.T for cb in (cb0, cb1, cb2, cb3)]
    mesh = plsc.VectorSubcoreMesh(core_axis_name="c", subcore_axis_name="s")
    f = functools.partial(
        pl.kernel,
        mesh=mesh,
        compiler_params=pltpu.CompilerParams(needs_layout_passes=False),
        out_type=jax.ShapeDtypeStruct((FEAT, BATCH), jnp.float32),
        scratch_types=[
            pltpu.VMEM((BATCH,), jnp.int32),
            pltpu.VMEM((BATCH,), jnp.float32),
            pltpu.VMEM((4096,), jnp.float32),
            pltpu.VMEM((16384,), jnp.float32),
            pltpu.VMEM((CHUNK,), jnp.float32),
            pltpu.VMEM((CHUNK,), jnp.float32),
            pltpu.SemaphoreType.DMA,
            pltpu.SemaphoreType.DMA,
            pltpu.SemaphoreType.DMA,
        ],
    )(_body)
    out_t = f(indices, *ts)
    return out_t.T


def kernel(indices, cb0, cb1, cb2, cb3):
    return _octree_lookup(indices.astype(jnp.int32), cb0, cb1, cb2, cb3)
